# baseline (speedup 1.0000x reference)
import functools

import jax
import jax.numpy as jnp
from jax import lax
from jax.experimental import pallas as pl
from jax.experimental.pallas import tpu as pltpu
from jax.experimental.pallas import tpu_sc as plsc

B = 16384
D = 64
NE = 1000000
NR = 1000
MARGIN = 1.0

NC = 2
NS = 16
NW = NC * NS
PER_W = B // NW
CHUNK = 128
NCHUNK = PER_W // CHUNK
NARR = 6
NITEM = NARR * NCHUNK
NBUF = 6

ENT_BLK = 8192
REL_BLK = 256
ENT_GRID = -(-NE // (4 * ENT_BLK))
REL_GRID = -(-NR // (4 * REL_BLK))

_mesh = plsc.VectorSubcoreMesh(core_axis_name="c", subcore_axis_name="s")


def _tp_body(b0, b1, b2, b3, out_ref):
    a = jnp.concatenate([b0[...], b2[...]], axis=0).T
    bb = jnp.concatenate([b1[...], b3[...]], axis=0).T
    ua = lax.bitcast_convert_type(a.astype(jnp.bfloat16),
                                  jnp.uint16).astype(jnp.uint32)
    ub = lax.bitcast_convert_type(bb.astype(jnp.bfloat16),
                                  jnp.uint16).astype(jnp.uint32)
    out_ref[...] = lax.bitcast_convert_type(ua | (ub << 16), jnp.float32)


def _make_transpose(grid, blk, n_cols):
    last = (n_cols - 1) // blk
    return pl.pallas_call(
        _tp_body,
        grid=(grid,),
        in_specs=[
            pl.BlockSpec((D, blk), lambda i, k=k, last=last:
                         (0, jnp.minimum(4 * i + k, last)))
            for k in range(4)
        ],
        out_specs=pl.BlockSpec((blk, 2 * D), lambda i: (i, 0)),
        out_shape=jax.ShapeDtypeStruct((grid * blk, 2 * D), jnp.float32),
        compiler_params=pltpu.CompilerParams(
            dimension_semantics=("parallel",)),
    )


_tp_ent = _make_transpose(ENT_GRID, ENT_BLK, NE)
_tp_rel = _make_transpose(REL_GRID, REL_BLK, NR)


def _gather_body(narr, tables, idx_hbms, outs, idx_v, rows_v, sems):
    nitem = narr * NCHUNK
    gsem = sems[:NBUF]
    wsem = sems[NBUF:2 * NBUF]
    psem = sems[2 * NBUF]
    wid = lax.axis_index("s") * NC + lax.axis_index("c")
    base = wid * PER_W

    ph_handles = [
        pltpu.async_copy(idx_hbms[a].at[pl.ds(base, PER_W)], idx_v.at[a],
                         psem)
        for a in range(narr)
    ]
    for h in ph_handles:
        h.wait()

    def start_gather(k):
        a, c = divmod(k, NCHUNK)
        return pltpu.async_copy(
            tables[a].at[idx_v.at[a, pl.ds(c * CHUNK, CHUNK)]],
            rows_v.at[k % NBUF], gsem[k % NBUF])

    def start_wb(k):
        a, c = divmod(k, NCHUNK)
        return pltpu.async_copy(
            rows_v.at[k % NBUF],
            outs[a].at[pl.ds(base + c * CHUNK, CHUNK)], wsem[k % NBUF])

    gh = {k: start_gather(k) for k in range(min(NBUF - 1, nitem))}
    wh = {}
    for k in range(nitem):
        gh[k].wait()
        wh[k] = start_wb(k)
        nxt = k + NBUF - 1
        if nxt < nitem:
            if k >= 1:
                wh[k - 1].wait()
            gh[nxt] = start_gather(nxt)
    for k in range(max(0, nitem - NBUF), nitem):
        wh[k].wait()


@functools.partial(
    pl.kernel,
    out_type=[jax.ShapeDtypeStruct((B, 2 * D), jnp.float32)
              for _ in range(4)],
    mesh=_mesh,
    scratch_types=[
        pltpu.VMEM((4, PER_W), jnp.int32),
        pltpu.VMEM((NBUF, CHUNK, 2 * D), jnp.float32),
    ] + [pltpu.SemaphoreType.DMA] * (2 * NBUF + 1),
    compiler_params=pltpu.CompilerParams(use_tc_tiling_on_sc=True),
)
def _sc_gather_ent(ent_hbm, ph_hbm, pt_hbm, nh_hbm, nt_hbm,
                   o_ph, o_pt, o_nh, o_nt, idx_v, rows_v, *sems):
    _gather_body(4, (ent_hbm,) * 4, (ph_hbm, pt_hbm, nh_hbm, nt_hbm),
                 (o_ph, o_pt, o_nh, o_nt), idx_v, rows_v, sems)


@functools.partial(
    pl.kernel,
    out_type=[jax.ShapeDtypeStruct((B, 2 * D), jnp.float32)
              for _ in range(2)],
    mesh=_mesh,
    scratch_types=[
        pltpu.VMEM((2, PER_W), jnp.int32),
        pltpu.VMEM((NBUF, CHUNK, 2 * D), jnp.float32),
    ] + [pltpu.SemaphoreType.DMA] * (2 * NBUF + 1),
    compiler_params=pltpu.CompilerParams(use_tc_tiling_on_sc=True),
)
def _sc_gather_rel(rel_hbm, pr_hbm, nr_hbm, o_pr, o_nr, idx_v, rows_v,
                   *sems):
    _gather_body(2, (rel_hbm,) * 2, (pr_hbm, nr_hbm), (o_pr, o_nr),
                 idx_v, rows_v, sems)


BLK = 2048
GRID = B // BLK


def _sel(pair_ref, idx_ref, half_bit):
    idx = idx_ref[...]
    lane_hi = (idx & (1 << half_bit)) != 0
    u = lax.bitcast_convert_type(
        jnp.where(lane_hi, pair_ref[:, D:], pair_ref[:, :D]), jnp.uint32)
    shift = ((idx >> (half_bit - 5)) & 16).astype(jnp.uint32)
    return lax.bitcast_convert_type((u >> shift) << 16, jnp.float32)


def _tc_body(ph, pr, pt, nh, nr, nt, iph, ipr, ipt, inh, inr, int_, out):
    i = pl.program_id(0)
    pos = _sel(ph, iph, 14) + _sel(pr, ipr, 9) - _sel(pt, ipt, 14)
    neg = _sel(nh, inh, 14) + _sel(nr, inr, 9) - _sel(nt, int_, 14)
    pd = jnp.sqrt(jnp.sum(pos * pos, axis=1))
    nd = jnp.sqrt(jnp.sum(neg * neg, axis=1))
    part = jnp.sum(jnp.maximum(pd - nd + MARGIN, 0.0))

    @pl.when(i == 0)
    def _():
        out[0] = 0.0

    out[0] += part

    @pl.when(i == GRID - 1)
    def _():
        out[0] = out[0] / B


_tc_loss = pl.pallas_call(
    _tc_body,
    grid=(GRID,),
    in_specs=[pl.BlockSpec((BLK, 2 * D), lambda i: (i, 0))] * 6
    + [pl.BlockSpec((BLK, 1), lambda i: (i, 0))] * 6,
    out_specs=pl.BlockSpec(memory_space=pltpu.SMEM),
    out_shape=jax.ShapeDtypeStruct((1,), jnp.float32),
)


def _pack_idx(i, blk):
    return (i // (4 * blk)) * blk + (i % blk)


def kernel(pos_h, pos_r, pos_t, neg_h, neg_r, neg_t, entity_emb,
           relation_emb):
    idx = [a.astype(jnp.int32) for a in
           (pos_h, pos_r, pos_t, neg_h, neg_r, neg_t)]
    blks = (ENT_BLK, REL_BLK, ENT_BLK, ENT_BLK, REL_BLK, ENT_BLK)
    packed_idx = [_pack_idx(a, blk) for a, blk in zip(idx, blks)]
    raw_idx = [a.reshape(B, 1) for a in idx]
    ent_t = entity_emb.T
    rel_t = relation_emb.T
    rel2 = _tp_rel(rel_t, rel_t, rel_t, rel_t)
    g_pr, g_nr = _sc_gather_rel(rel2, packed_idx[1], packed_idx[4])
    ent2 = _tp_ent(ent_t, ent_t, ent_t, ent_t)
    g_ph, g_pt, g_nh, g_nt = _sc_gather_ent(
        ent2, packed_idx[0], packed_idx[2], packed_idx[3], packed_idx[5])
    loss = _tc_loss(g_ph, g_pr, g_pt, g_nh, g_nr, g_nt, *raw_idx)
    return loss[0]


# speedup vs baseline: 1.0187x; 1.0187x over previous
import functools

import jax
import jax.numpy as jnp
from jax import lax
from jax.experimental import pallas as pl
from jax.experimental.pallas import tpu as pltpu
from jax.experimental.pallas import tpu_sc as plsc

B = 16384
D = 64
NE = 1000000
NR = 1000
MARGIN = 1.0

NC = 2
NS = 16
NW = NC * NS
PER_W = B // NW
CHUNK = 128
NCHUNK = PER_W // CHUNK
NARR = 6
NITEM = NARR * NCHUNK
NBUF = 6

ENT_BLK = 8192
REL_BLK = 256
ENT_GRID = -(-NE // (4 * ENT_BLK))
REL_GRID = -(-NR // (4 * REL_BLK))

_mesh = plsc.VectorSubcoreMesh(core_axis_name="c", subcore_axis_name="s")


def _tp_body(b0, b1, b2, b3, out_ref):
    a = jnp.concatenate([b0[...], b2[...]], axis=0).T
    bb = jnp.concatenate([b1[...], b3[...]], axis=0).T
    ua = lax.bitcast_convert_type(a.astype(jnp.bfloat16),
                                  jnp.uint16).astype(jnp.uint32)
    ub = lax.bitcast_convert_type(bb.astype(jnp.bfloat16),
                                  jnp.uint16).astype(jnp.uint32)
    out_ref[...] = lax.bitcast_convert_type(ua | (ub << 16), jnp.float32)


def _make_transpose(grid, blk, n_cols):
    last = (n_cols - 1) // blk
    return pl.pallas_call(
        _tp_body,
        grid=(grid,),
        in_specs=[
            pl.BlockSpec((D, blk), lambda i, k=k, last=last:
                         (0, jnp.minimum(4 * i + k, last)))
            for k in range(4)
        ],
        out_specs=pl.BlockSpec((blk, 2 * D), lambda i: (i, 0)),
        out_shape=jax.ShapeDtypeStruct((grid * blk, 2 * D), jnp.float32),
        compiler_params=pltpu.CompilerParams(
            dimension_semantics=("parallel",)),
    )


_tp_ent = _make_transpose(ENT_GRID, ENT_BLK, NE)
_tp_rel = _make_transpose(REL_GRID, REL_BLK, NR)


@functools.partial(
    pl.kernel,
    out_type=[jax.ShapeDtypeStruct((B, 2 * D), jnp.float32)
              for _ in range(NARR)],
    mesh=_mesh,
    scratch_types=[
        pltpu.VMEM((NARR, PER_W), jnp.int32),
        pltpu.VMEM((NBUF, CHUNK, 2 * D), jnp.float32),
    ] + [pltpu.SemaphoreType.DMA] * (2 * NBUF + 1),
    compiler_params=pltpu.CompilerParams(use_tc_tiling_on_sc=True),
)
def _sc_gather(ent_hbm, rel_hbm, ph_hbm, pr_hbm, pt_hbm, nh_hbm, nr_hbm,
               nt_hbm, o_ph, o_pr, o_pt, o_nh, o_nr, o_nt, idx_v, rows_v,
               *sems):
    gsem = sems[:NBUF]
    wsem = sems[NBUF:2 * NBUF]
    psem = sems[2 * NBUF]
    wid = lax.axis_index("s") * NC + lax.axis_index("c")
    base = wid * PER_W
    tables = (ent_hbm, rel_hbm, ent_hbm, ent_hbm, rel_hbm, ent_hbm)
    idx_hbms = (ph_hbm, pr_hbm, pt_hbm, nh_hbm, nr_hbm, nt_hbm)
    outs = (o_ph, o_pr, o_pt, o_nh, o_nr, o_nt)

    ph_handles = [
        pltpu.async_copy(idx_hbms[a].at[pl.ds(base, PER_W)], idx_v.at[a],
                         psem)
        for a in range(NARR)
    ]
    for h in ph_handles:
        h.wait()

    def start_gather(k):
        a, c = divmod(k, NCHUNK)
        return pltpu.async_copy(
            tables[a].at[idx_v.at[a, pl.ds(c * CHUNK, CHUNK)]],
            rows_v.at[k % NBUF], gsem[k % NBUF])

    def start_wb(k):
        a, c = divmod(k, NCHUNK)
        return pltpu.async_copy(
            rows_v.at[k % NBUF],
            outs[a].at[pl.ds(base + c * CHUNK, CHUNK)], wsem[k % NBUF])

    gh = {k: start_gather(k) for k in range(min(NBUF - 1, NITEM))}
    wh = {}
    for k in range(NITEM):
        gh[k].wait()
        wh[k] = start_wb(k)
        nxt = k + NBUF - 1
        if nxt < NITEM:
            if k >= 1:
                wh[k - 1].wait()
            gh[nxt] = start_gather(nxt)
    for k in range(max(0, NITEM - NBUF), NITEM):
        wh[k].wait()


BLK = 2048
GRID = B // BLK


def _sel(pair_ref, idx_ref, half_bit):
    idx = idx_ref[...]
    lane_hi = (idx & (1 << half_bit)) != 0
    u = lax.bitcast_convert_type(
        jnp.where(lane_hi, pair_ref[:, D:], pair_ref[:, :D]), jnp.uint32)
    shift = ((idx >> (half_bit - 5)) & 16).astype(jnp.uint32)
    return lax.bitcast_convert_type((u >> shift) << 16, jnp.float32)


def _tc_body(ph, pr, pt, nh, nr, nt, iph, ipr, ipt, inh, inr, int_, out):
    i = pl.program_id(0)
    pos = _sel(ph, iph, 14) + _sel(pr, ipr, 9) - _sel(pt, ipt, 14)
    neg = _sel(nh, inh, 14) + _sel(nr, inr, 9) - _sel(nt, int_, 14)
    pd = jnp.sqrt(jnp.sum(pos * pos, axis=1))
    nd = jnp.sqrt(jnp.sum(neg * neg, axis=1))
    part = jnp.sum(jnp.maximum(pd - nd + MARGIN, 0.0))

    @pl.when(i == 0)
    def _():
        out[0] = 0.0

    out[0] += part

    @pl.when(i == GRID - 1)
    def _():
        out[0] = out[0] / B


_tc_loss = pl.pallas_call(
    _tc_body,
    grid=(GRID,),
    in_specs=[pl.BlockSpec((BLK, 2 * D), lambda i: (i, 0))] * 6
    + [pl.BlockSpec((BLK, 1), lambda i: (i, 0))] * 6,
    out_specs=pl.BlockSpec(memory_space=pltpu.SMEM),
    out_shape=jax.ShapeDtypeStruct((1,), jnp.float32),
)


def _pack_idx(i, blk):
    return (i // (4 * blk)) * blk + (i % blk)


def kernel(pos_h, pos_r, pos_t, neg_h, neg_r, neg_t, entity_emb,
           relation_emb):
    idx = [a.astype(jnp.int32) for a in
           (pos_h, pos_r, pos_t, neg_h, neg_r, neg_t)]
    blks = (ENT_BLK, REL_BLK, ENT_BLK, ENT_BLK, REL_BLK, ENT_BLK)
    packed_idx = [_pack_idx(a, blk) for a, blk in zip(idx, blks)]
    raw_idx = [a.reshape(B, 1) for a in idx]
    ent_t = entity_emb.T
    rel_t = relation_emb.T
    ent2 = _tp_ent(ent_t, ent_t, ent_t, ent_t)
    rel2 = _tp_rel(rel_t, rel_t, rel_t, rel_t)
    pairs = _sc_gather(ent2, rel2, *packed_idx)
    loss = _tc_loss(*pairs, *raw_idx)
    return loss[0]
